# baseline (speedup 1.0000x reference)
import jax
import jax.numpy as jnp
from jax import lax
from jax.experimental import pallas as pl
from jax.experimental.pallas import tpu as pltpu
from jax.experimental.pallas import tpu_sc as plsc

B, C, H, W = 4, 64, 128, 128
HW = H * W
SPHERE = 32768
V = 16
P = 8192

NW = 32
PTS_PER_W = (B * P) // NW
CHUNKS_PER_B = P // PTS_PER_W
BLK_PTS = 4
NBLK = PTS_PER_W // BLK_PTS
RB = BLK_PTS * V
RING = 4


def _sc_body(xT, inds, vm, out, pinds, vmraw, idx2, rows, outb,
             gsem, xs0, xs1, xs2, xs3):
    xsems = (xs0, xs1, xs2, xs3)
    nc = 2
    wid = lax.axis_index("s") * nc + lax.axis_index("c")
    b = wid // CHUNKS_PER_B
    chunk = wid % CHUNKS_PER_B
    base = chunk * PTS_PER_W

    pltpu.sync_copy(inds.at[b, pl.ds(base, PTS_PER_W)], pinds)

    lane = lax.iota(jnp.int32, 16)
    ev = lane * 2
    od = ev + 1
    boff = b * HW
    ch_rows = [lane + c4 * 16 for c4 in range(4)]

    descs = [
        pltpu.async_copy(
            vm.at[pinds.at[pl.ds(j * 128, 128)]],
            vmraw.at[pl.ds(j * 128, 128)], gsem)
        for j in range(PTS_PER_W // 128)
    ]
    for d_ in descs:
        d_.wait()

    def build_and_fire(blk, d):
        @pl.loop(0, BLK_PTS)
        def _bq(q):
            p = blk * BLK_PTS + q
            prow = jnp.full((16,), p, jnp.int32)
            fidx = plsc.load_gather(vmraw, [prow, ev])
            idx2[d, pl.ds(q * V, V)] = fidx.astype(jnp.int32) + boff

        pltpu.async_copy(xT.at[idx2.at[d]], rows.at[d], xsems[d])

    for d in range(RING):
        build_and_fire(d, d)

    @pl.loop(0, NBLK, step=RING)
    def _main(g):
        for d in range(RING):
            blk = g + d
            pltpu.make_async_copy(
                xT.at[idx2.at[d]], rows.at[d], xsems[d]).wait()

            @pl.loop(0, BLK_PTS)
            def _pt(q):
                p = blk * BLK_PTS + q
                prow = jnp.full((16,), p, jnp.int32)
                wrow = plsc.load_gather(vmraw, [prow, od])
                accA = [jnp.zeros((16,), jnp.float32) for _ in range(4)]
                accB = [jnp.zeros((16,), jnp.float32) for _ in range(4)]
                for v in range(V):
                    w = wrow[v]
                    r = q * V + v
                    acc = accA if v % 2 == 0 else accB
                    for c4 in range(4):
                        acc[c4] = acc[c4] + w * rows[d, r, pl.ds(c4 * 16, 16)]
                pcol = jnp.full((16,), p, jnp.int32)
                for c4 in range(4):
                    plsc.store_scatter(outb, [ch_rows[c4], pcol],
                                       accA[c4] + accB[c4])

            nxt = blk + RING

            @pl.when(nxt < NBLK)
            def _():
                build_and_fire(nxt, d)

    pltpu.sync_copy(outb, out.at[b, :, pl.ds(base, PTS_PER_W)])


def kernel(x, inds, vote_mapping):
    xT = jnp.transpose(x, (0, 2, 3, 1)).reshape(B * HW, C)
    vm = vote_mapping.reshape(SPHERE, 2 * V)
    mesh = plsc.VectorSubcoreMesh(core_axis_name="c", subcore_axis_name="s")
    f = pl.kernel(
        _sc_body,
        out_type=jax.ShapeDtypeStruct((B, C, P), jnp.float32),
        mesh=mesh,
        scratch_types=[
            pltpu.VMEM((PTS_PER_W,), jnp.int32),
            pltpu.VMEM((PTS_PER_W, 2 * V), jnp.float32),
            pltpu.VMEM((RING, RB), jnp.int32),
            pltpu.VMEM((RING, RB, C), jnp.float32),
            pltpu.VMEM((C, PTS_PER_W), jnp.float32),
            pltpu.SemaphoreType.DMA,
            pltpu.SemaphoreType.DMA,
            pltpu.SemaphoreType.DMA,
            pltpu.SemaphoreType.DMA,
            pltpu.SemaphoreType.DMA,
        ],
        compiler_params=pltpu.CompilerParams(
            needs_layout_passes=False, use_tc_tiling_on_sc=False),
    )
    return f(xT, inds, vm)


# speedup vs baseline: 1.1464x; 1.1464x over previous
import jax
import jax.numpy as jnp
from jax import lax
from jax.experimental import pallas as pl
from jax.experimental.pallas import tpu as pltpu
from jax.experimental.pallas import tpu_sc as plsc

B, C, H, W = 4, 64, 128, 128
HW = H * W
SPHERE = 32768
V = 16
P = 8192

NW = 32
PTS_PER_W = (B * P) // NW
CHUNKS_PER_B = P // PTS_PER_W
BLK_PTS = 4
NBLK = PTS_PER_W // BLK_PTS
RB = BLK_PTS * V
RING = 4


def _sc_body(xT, inds, vm, out, pinds, vmraw, idx2, rows, outb,
             gsem, xs0, xs1, xs2, xs3):
    xsems = (xs0, xs1, xs2, xs3)
    nc = 2
    wid = lax.axis_index("s") * nc + lax.axis_index("c")
    b = wid // CHUNKS_PER_B
    chunk = wid % CHUNKS_PER_B
    base = chunk * PTS_PER_W

    pltpu.sync_copy(inds.at[b, pl.ds(base, PTS_PER_W)], pinds)

    lane = lax.iota(jnp.int32, 16)
    ev = lane * 2
    od = ev + 1
    boff = b * HW
    ch_rows = [lane + c4 * 16 for c4 in range(4)]

    descs = [
        pltpu.async_copy(
            vm.at[pinds.at[pl.ds(j * 128, 128)]],
            vmraw.at[pl.ds(j * 128, 128)], gsem)
        for j in range(PTS_PER_W // 128)
    ]
    for d_ in descs:
        d_.wait()

    def build_and_fire(blk, d):
        @pl.loop(0, BLK_PTS)
        def _bq(q):
            p = blk * BLK_PTS + q
            prow = jnp.full((16,), p, jnp.int32)
            fidx = plsc.load_gather(vmraw, [prow, ev])
            idx2[d, pl.ds(q * V, V)] = fidx.astype(jnp.int32) + boff

        pltpu.async_copy(xT.at[idx2.at[d]], rows.at[d], xsems[d])

    for d in range(RING):
        build_and_fire(d, d)

    @pl.loop(0, NBLK, step=RING)
    def _main(g):
        for d in range(RING):
            blk = g + d
            pltpu.make_async_copy(
                xT.at[idx2.at[d]], rows.at[d], xsems[d]).wait()

            @pl.loop(0, BLK_PTS)
            def _pt(q):
                p = blk * BLK_PTS + q
                prow = jnp.full((16,), p, jnp.int32)
                wrow = plsc.load_gather(vmraw, [prow, od])
                accA = [jnp.zeros((16,), jnp.float32) for _ in range(4)]
                accB = [jnp.zeros((16,), jnp.float32) for _ in range(4)]
                for v in range(V):
                    w = wrow[v]
                    r = q * V + v
                    acc = accA if v % 2 == 0 else accB
                    for c4 in range(4):
                        acc[c4] = acc[c4] + w * rows[d, r, pl.ds(c4 * 16, 16)]
                pcol = jnp.full((16,), p, jnp.int32)
                for c4 in range(4):
                    plsc.store_scatter(outb, [ch_rows[c4], pcol],
                                       accA[c4] + accB[c4])

            nxt = blk + RING

            @pl.when(nxt < NBLK)
            def _():
                build_and_fire(nxt, d)

    pltpu.sync_copy(outb.at[:, pl.ds(0, PTS_PER_W)],
                    out.at[b, :, pl.ds(base, PTS_PER_W)])


def kernel(x, inds, vote_mapping):
    xT = jnp.transpose(x, (0, 2, 3, 1)).reshape(B * HW, C)
    vm = vote_mapping.reshape(SPHERE, 2 * V)
    mesh = plsc.VectorSubcoreMesh(core_axis_name="c", subcore_axis_name="s")
    f = pl.kernel(
        _sc_body,
        out_type=jax.ShapeDtypeStruct((B, C, P), jnp.float32),
        mesh=mesh,
        scratch_types=[
            pltpu.VMEM((PTS_PER_W,), jnp.int32),
            pltpu.VMEM((PTS_PER_W, 2 * V), jnp.float32),
            pltpu.VMEM((RING, RB), jnp.int32),
            pltpu.VMEM((RING, RB, C), jnp.float32),
            pltpu.VMEM((C, PTS_PER_W + 1), jnp.float32),
            pltpu.SemaphoreType.DMA,
            pltpu.SemaphoreType.DMA,
            pltpu.SemaphoreType.DMA,
            pltpu.SemaphoreType.DMA,
            pltpu.SemaphoreType.DMA,
        ],
        compiler_params=pltpu.CompilerParams(
            needs_layout_passes=False, use_tc_tiling_on_sc=False),
    )
    return f(xT, inds, vm)
